# baseline (device time: 23710 ns/iter reference)
import jax
import jax.numpy as jnp
from jax import lax
from jax.experimental import pallas as pl
from jax.experimental.pallas import tpu as pltpu

K = 16
NEG_INF = float("-inf")
N_CHUNKS = 4


def _topk_desc(xv, k):
    cols = []
    for _ in range(k):
        m = jnp.max(xv, axis=1, keepdims=True)
        cols.append(m)
        xv = jnp.where(xv == m, NEG_INF, xv)
    return jnp.concatenate(cols, axis=1)


def _lane_top4(xc):
    out = []
    for _ in range(4):
        li = jnp.max(xc, axis=1)
        out.append(li)
        xc = jnp.where(xc == li[:, None, :], NEG_INF, xc)
    return out


def _merge_top4(a, b):
    u = [jnp.maximum(a[i], b[3 - i]) for i in range(4)]
    v0, v2 = jnp.maximum(u[0], u[2]), jnp.minimum(u[0], u[2])
    v1, v3 = jnp.maximum(u[1], u[3]), jnp.minimum(u[1], u[3])
    return [
        jnp.maximum(v0, v1), jnp.minimum(v0, v1),
        jnp.maximum(v2, v3), jnp.minimum(v2, v3),
    ]


def kernel(x):
    rows, n = x.shape
    half = rows // 2
    chunk = n // N_CHUNKS

    def body(x_hbm, out_ref, buf, cand_ref, copy_sems,
             send_sem_x, recv_sem_x, send_sem_y, recv_sem_y):
        my_x = lax.axis_index("x")
        my_y = lax.axis_index("y")
        nbr_x = (1 - my_x, my_y)
        nbr_y = (my_x, 1 - my_y)
        row0 = my_y * half

        barrier_sem = pltpu.get_barrier_semaphore()
        for nbr in (nbr_x, nbr_y):
            pl.semaphore_signal(
                barrier_sem, inc=1, device_id=nbr,
                device_id_type=pl.DeviceIdType.MESH,
            )
        pl.semaphore_wait(barrier_sem, 2)

        def chunk_copy(c, slot):
            return pltpu.make_async_copy(
                x_hbm.at[pl.ds(row0, half), pl.ds(c * chunk, chunk)],
                buf.at[slot],
                copy_sems.at[slot],
            )

        chunk_copy(0, 0).start()
        s = None
        for c in range(N_CHUNKS):
            slot = c % 2
            if c + 1 < N_CHUNKS:
                chunk_copy(c + 1, (c + 1) % 2).start()
            chunk_copy(c, slot).wait()
            xc = buf[slot].reshape(half, chunk // 128, 128)
            t4 = _lane_top4(xc)
            s = t4 if s is None else _merge_top4(s, t4)

        cands = jnp.concatenate([t[:, None, :] for t in s], axis=1)
        cand_ref[0] = _topk_desc(cands.reshape(half, 4 * 128), K)

        rdma_x = pltpu.make_async_remote_copy(
            src_ref=cand_ref.at[0],
            dst_ref=cand_ref.at[1],
            send_sem=send_sem_x,
            recv_sem=recv_sem_x,
            device_id=nbr_x,
            device_id_type=pl.DeviceIdType.MESH,
        )
        rdma_x.start()
        rdma_x.wait()
        merged = jnp.concatenate([cand_ref[0], cand_ref[1]], axis=1)
        out_ref[pl.ds(row0, half), :] = _topk_desc(merged, K)

        rdma_y = pltpu.make_async_remote_copy(
            src_ref=out_ref.at[pl.ds(row0, half), :],
            dst_ref=out_ref.at[pl.ds(row0, half), :],
            send_sem=send_sem_y,
            recv_sem=recv_sem_y,
            device_id=nbr_y,
            device_id_type=pl.DeviceIdType.MESH,
        )
        rdma_y.start()
        rdma_y.wait()

    return pl.pallas_call(
        body,
        out_shape=jax.ShapeDtypeStruct((rows, K), jnp.float32),
        in_specs=[pl.BlockSpec(memory_space=pl.ANY)],
        out_specs=pl.BlockSpec(memory_space=pltpu.VMEM),
        scratch_shapes=[
            pltpu.VMEM((2, half, chunk), jnp.float32),
            pltpu.VMEM((2, half, K), jnp.float32),
            pltpu.SemaphoreType.DMA((2,)),
            pltpu.SemaphoreType.DMA,
            pltpu.SemaphoreType.DMA,
            pltpu.SemaphoreType.DMA,
            pltpu.SemaphoreType.DMA,
        ],
        compiler_params=pltpu.CompilerParams(collective_id=0),
    )(x)


# device time: 19604 ns/iter; 1.2094x vs baseline; 1.2094x over previous
import jax
import jax.numpy as jnp
from jax import lax
from jax.experimental import pallas as pl
from jax.experimental.pallas import tpu as pltpu

K = 16
NEG_INF = float("-inf")
N_BLK = 2


def _topk_cols(xv, k):
    cols = []
    for _ in range(k):
        m = jnp.max(xv, axis=1, keepdims=True)
        cols.append(m)
        xv = jnp.where(xv == m, NEG_INF, xv)
    return cols


def _block_topk(xv):
    rows, n = xv.shape
    x3 = xv.reshape(rows, n // 128, 128)
    cands = []
    for _ in range(4):
        li = jnp.max(x3, axis=1)
        cands.append(li)
        x3 = jnp.where(x3 == li[:, None, :], NEG_INF, x3)
    cols = _topk_cols(jnp.concatenate(cands, axis=1), K)
    return jnp.concatenate(cols, axis=1), jnp.concatenate(cols[::-1], axis=1)


def _bitonic_desc(u):
    rows = u.shape[0]
    for d in (8, 4, 2, 1):
        g = u.reshape(rows, 16 // (2 * d), 2, d)
        hi = jnp.maximum(g[:, :, 0, :], g[:, :, 1, :])
        lo = jnp.minimum(g[:, :, 0, :], g[:, :, 1, :])
        u = jnp.stack([hi, lo], axis=2).reshape(rows, 16)
    return u


def _merge16(a_desc, b_asc):
    return _bitonic_desc(jnp.maximum(a_desc, b_asc))


def kernel(x):
    rows, n = x.shape
    half = rows // 2
    blk = half // N_BLK

    def body(x_ref, out_ref, loc_ref, asc_ref, rem_ref,
             sx_sems, rx_sems, sy_sems, ry_sems):
        my_x = lax.axis_index("x")
        my_y = lax.axis_index("y")
        nbr_x = (1 - my_x, my_y)
        nbr_y = (my_x, 1 - my_y)
        row0 = my_y * half

        barrier_sem = pltpu.get_barrier_semaphore()
        for nbr in (nbr_x, nbr_y):
            pl.semaphore_signal(
                barrier_sem, inc=1, device_id=nbr,
                device_id_type=pl.DeviceIdType.MESH,
            )
        pl.semaphore_wait(barrier_sem, 2)

        def rdma_x(b):
            return pltpu.make_async_remote_copy(
                src_ref=asc_ref.at[b],
                dst_ref=rem_ref.at[b],
                send_sem=sx_sems.at[b],
                recv_sem=rx_sems.at[b],
                device_id=nbr_x,
                device_id_type=pl.DeviceIdType.MESH,
            )

        def rdma_y(b):
            sl = pl.ds(row0 + b * blk, blk)
            return pltpu.make_async_remote_copy(
                src_ref=out_ref.at[sl, :],
                dst_ref=out_ref.at[sl, :],
                send_sem=sy_sems.at[b],
                recv_sem=ry_sems.at[b],
                device_id=nbr_y,
                device_id_type=pl.DeviceIdType.MESH,
            )

        for b in range(N_BLK):
            desc, asc = _block_topk(x_ref[pl.ds(row0 + b * blk, blk), :])
            loc_ref[b] = desc
            asc_ref[b] = asc
            rdma_x(b).start()

        for b in range(N_BLK):
            rdma_x(b).wait()
            out_ref[pl.ds(row0 + b * blk, blk), :] = _merge16(
                loc_ref[b], rem_ref[b]
            )
            rdma_y(b).start()
        for b in range(N_BLK):
            rdma_y(b).wait()

    return pl.pallas_call(
        body,
        out_shape=jax.ShapeDtypeStruct((rows, K), jnp.float32),
        in_specs=[pl.BlockSpec(memory_space=pltpu.VMEM)],
        out_specs=pl.BlockSpec(memory_space=pltpu.VMEM),
        scratch_shapes=[
            pltpu.VMEM((N_BLK, blk, K), jnp.float32),
            pltpu.VMEM((N_BLK, blk, K), jnp.float32),
            pltpu.VMEM((N_BLK, blk, K), jnp.float32),
            pltpu.SemaphoreType.DMA((N_BLK,)),
            pltpu.SemaphoreType.DMA((N_BLK,)),
            pltpu.SemaphoreType.DMA((N_BLK,)),
            pltpu.SemaphoreType.DMA((N_BLK,)),
        ],
        compiler_params=pltpu.CompilerParams(collective_id=0),
    )(x)
